# Initial kernel scaffold; baseline (speedup 1.0000x reference)
import jax
import jax.numpy as jnp
from jax.experimental import pallas as pl


def kernel(x, y, centroids):
    raise NotImplementedError("write your pallas kernel here")


# speedup vs baseline: 1.5140x; 1.5140x over previous
import jax
import jax.numpy as jnp
from jax.experimental import pallas as pl
from jax.experimental.pallas import tpu as pltpu

B, D, C = 2048, 1024, 8192
BB = 1024
BC = 512


def _fused_kernel(x_ref, ct_ref, y_ref, loss_ref, corr_ref,
                  m_ref, l_ref, t_ref, a_ref):
    c = pl.program_id(1)
    nc = pl.num_programs(1)

    @pl.when(c == 0)
    def _init():
        m_ref[...] = jnp.full(m_ref.shape, -jnp.inf, dtype=jnp.float32)
        l_ref[...] = jnp.zeros(l_ref.shape, dtype=jnp.float32)
        t_ref[...] = jnp.zeros(t_ref.shape, dtype=jnp.float32)
        a_ref[...] = jnp.zeros(a_ref.shape, dtype=jnp.int32)

    xb = x_ref[...]
    ctb = ct_ref[...]
    acc = jnp.dot(xb, ctb, preferred_element_type=jnp.float32)
    c2 = jnp.sum(ctb * ctb, axis=0, keepdims=True)
    g = acc * 2.0 - c2

    cmax = jnp.max(g, axis=1, keepdims=True)
    col = jax.lax.broadcasted_iota(jnp.int32, (BB, BC), 1)
    camax = jnp.min(jnp.where(g >= cmax, col, C), axis=1, keepdims=True) + c * BC
    y_col = y_ref[...]
    tsum = jnp.sum(jnp.where(col + c * BC == y_col, g, 0.0),
                   axis=1, keepdims=True)

    m_old = jnp.max(m_ref[...], axis=1, keepdims=True)
    l_old = jnp.max(l_ref[...], axis=1, keepdims=True)
    a_old = jnp.max(a_ref[...], axis=1, keepdims=True)

    m_new = jnp.maximum(m_old, cmax)
    p_sum = jnp.sum(jnp.exp(g - m_new), axis=1, keepdims=True)
    l_new = l_old * jnp.exp(m_old - m_new) + p_sum
    a_new = jnp.where(cmax > m_old, camax, a_old)

    m_ref[...] = jnp.broadcast_to(m_new, m_ref.shape)
    l_ref[...] = jnp.broadcast_to(l_new, l_ref.shape)
    a_ref[...] = jnp.broadcast_to(a_new, a_ref.shape)
    t_ref[...] = t_ref[...] + jnp.broadcast_to(tsum, t_ref.shape)

    @pl.when(c == nc - 1)
    def _fin():
        m_c = jnp.max(m_ref[...], axis=1, keepdims=True)
        l_c = jnp.max(l_ref[...], axis=1, keepdims=True)
        t_c = jnp.max(t_ref[...], axis=1, keepdims=True)
        a_c = jnp.max(a_ref[...], axis=1, keepdims=True)
        lse = m_c + jnp.log(l_c)
        loss_col = lse - t_c
        corr_col = (a_c == y_ref[...]).astype(jnp.float32)
        ls = jnp.sum(loss_col, keepdims=True)
        cs = jnp.sum(corr_col, keepdims=True)
        loss_ref[...] = jnp.broadcast_to(ls, (8, 128)).reshape(1, 8, 128)
        corr_ref[...] = jnp.broadcast_to(cs, (8, 128)).reshape(1, 8, 128)


@jax.jit
def kernel(x, y, centroids):
    ct = centroids.T
    y_col = y.astype(jnp.int32).reshape(B, 1)
    nb = B // BB
    out_shape = (jax.ShapeDtypeStruct((nb, 8, 128), jnp.float32),
                 jax.ShapeDtypeStruct((nb, 8, 128), jnp.float32))
    loss_p, corr_p = pl.pallas_call(
        _fused_kernel,
        grid=(nb, C // BC),
        in_specs=[
            pl.BlockSpec((BB, D), lambda b, c: (b, 0)),
            pl.BlockSpec((D, BC), lambda b, c: (0, c)),
            pl.BlockSpec((BB, 1), lambda b, c: (b, 0)),
        ],
        out_specs=(pl.BlockSpec((1, 8, 128), lambda b, c: (b, 0, 0)),
                   pl.BlockSpec((1, 8, 128), lambda b, c: (b, 0, 0))),
        out_shape=out_shape,
        scratch_shapes=[
            pltpu.VMEM((BB, 128), jnp.float32),
            pltpu.VMEM((BB, 128), jnp.float32),
            pltpu.VMEM((BB, 128), jnp.float32),
            pltpu.VMEM((BB, 128), jnp.int32),
        ],
        compiler_params=pltpu.CompilerParams(
            dimension_semantics=("parallel", "arbitrary"),
            vmem_limit_bytes=100 * 1024 * 1024,
        ),
    )(x, ct, y_col)
    loss = jnp.sum(loss_p[:, 0, 0]) / B
    score = jnp.sum(corr_p[:, 0, 0]) / B
    return loss, score
